# baseline (speedup 1.0000x reference)
import functools

import jax
import jax.numpy as jnp
from jax import lax
from jax.experimental import pallas as pl
from jax.experimental.pallas import tpu as pltpu
from jax.experimental.pallas import tpu_sc as plsc

N = 16777216
NC, NS, L = 2, 16, 16
NW = NC * NS
PW = N // NW
C = 16384
NCHUNK = PW // C

_LN2 = 0.6931471805599453

_A = (3.0, 15.0, 30.0, -1.0)
_B = (0.0, -0.1, -0.4, 0.0)

_BAND_BY_SLOT = (3, 0, 0, 0, 3, 1, 1, 3, 2, 2, 2, 3, 0, 1, 2, 3)
_NEG = float("-inf")
_THR_TAB = (_NEG, 1.0e9, _NEG, _NEG, _NEG, 1.0e10, _NEG, _NEG,
            100000006144.0, _NEG, _NEG, 1.0e12, _NEG, _NEG, _NEG, _NEG)
_BASE_TAB = (13, 0, 10, 10, 10, 3, 11, 11, 6, 12, 12, 9, 13, 0, 0, 0)

_P4 = (-159.80915647634861, 5.371138987534238, -3.6996336354567507,
       1.4905116583281666, -0.2501516357927904)


def _table_inputs():
    rows = [
        _THR_TAB,
        [_A[i] for i in _BAND_BY_SLOT],
        [_B[i] * _LN2 for i in _BAND_BY_SLOT],
    ]
    return (jnp.asarray(rows, dtype=jnp.float32),
            jnp.asarray(_BASE_TAB, dtype=jnp.int32))


def _take(vec, idx):
    return jnp.take_along_axis(vec, idx, axis=0, mode="promise_in_bounds")


def _eval_vec(f, thr_v, base_v, a_v, bln_v):
    bits = lax.bitcast_convert_type(f, jnp.int32)

    idx_bin = (bits >> 23) - 155
    thr = _take(thr_v, idx_bin)
    one = jnp.ones_like(idx_bin)
    g = jnp.where(f > thr, 2 * one, jnp.where(f >= thr, one, 0 * one))
    slot = _take(base_v, idx_bin) + g
    a = _take(a_v, slot)
    bln = _take(bln_v, slot)

    ebr = (bits + 0x00400000) >> 23
    scale = lax.bitcast_convert_type((254 - ebr) << 23, jnp.float32)
    m = f * scale
    p = jnp.float32(_P4[4])
    p = p * m + jnp.float32(_P4[3])
    p = p * m + jnp.float32(_P4[2])
    p = p * m + jnp.float32(_P4[1])
    p = p * m + jnp.float32(_P4[0])
    lg2 = p + ebr.astype(jnp.float32)

    return a * jnp.exp(bln * lg2)


@functools.cache
def _build_sc_rel():
    @functools.partial(
        pl.kernel,
        out_type=jax.ShapeDtypeStruct((N,), jnp.float32),
        mesh=plsc.VectorSubcoreMesh(
            core_axis_name="c", subcore_axis_name="s",
            num_cores=NC, num_subcores=NS),
        scratch_types=[
            pltpu.VMEM((3, L), jnp.float32), pltpu.VMEM((L,), jnp.int32),
            pltpu.VMEM((C,), jnp.float32), pltpu.VMEM((C,), jnp.float32),
            pltpu.VMEM((C,), jnp.float32), pltpu.VMEM((C,), jnp.float32),
            pltpu.SemaphoreType.DMA, pltpu.SemaphoreType.DMA,
            pltpu.SemaphoreType.DMA, pltpu.SemaphoreType.DMA,
        ],
    )
    def _sc_rel(freq_hbm, ftab_hbm, btab_hbm, rel_hbm,
                ftab_v, btab_v, in0, in1, rel0, rel1,
                isem0, isem1, osem0, osem1):
        wid = lax.axis_index("s") * NC + lax.axis_index("c")
        base0 = wid * PW

        def in_slice(k):
            return freq_hbm.at[pl.ds(base0 + k * C, C)]

        pltpu.async_copy(in_slice(0), in0, isem0)
        pltpu.async_copy(in_slice(1), in1, isem1)
        pltpu.sync_copy(ftab_hbm, ftab_v)
        pltpu.sync_copy(btab_hbm, btab_v)

        thr_v = ftab_v[0]
        a_v = ftab_v[1]
        bln_v = ftab_v[2]
        base_v = btab_v[...]

        def compute(in_v, rel_v):
            @plsc.parallel_loop(0, C, L)
            def _(i):
                rel_v[pl.ds(i, L)] = _eval_vec(
                    in_v[pl.ds(i, L)], thr_v, base_v, a_v, bln_v)

        def half(kk, k, in_v, rel_v, isem, osem):
            base = base0 + k * C
            pltpu.make_async_copy(in_slice(k), in_v, isem).wait()

            @pl.when(kk > 0)
            def _():
                pltpu.make_async_copy(
                    rel_v, rel_hbm.at[pl.ds(base, C)], osem).wait()

            compute(in_v, rel_v)
            pltpu.async_copy(rel_v, rel_hbm.at[pl.ds(base, C)], osem)

            @pl.when(kk < NCHUNK // 2 - 1)
            def _():
                pltpu.async_copy(in_slice(k + 2), in_v, isem)

        def pair_body(kk, carry):
            half(kk, 2 * kk, in0, rel0, isem0, osem0)
            half(kk, 2 * kk + 1, in1, rel1, isem1, osem1)
            return carry

        lax.fori_loop(0, NCHUNK // 2, pair_body, 0)

        last0 = base0 + (NCHUNK - 2) * C
        last1 = base0 + (NCHUNK - 1) * C
        pltpu.make_async_copy(rel0, rel_hbm.at[pl.ds(last0, C)], osem0).wait()
        pltpu.make_async_copy(rel1, rel_hbm.at[pl.ds(last1, C)], osem1).wait()

    return _sc_rel


LANES = 1024
ROWS = N // LANES
BR = 1024
GRID = ROWS // BR


def _tc_body(f_ref, cond_ref):
    f = f_ref[...]
    x = f * jnp.float32(1e-9)
    b0 = (x >= 1.0) & (x < 10.0)
    b1 = (x > 10.0) & (x < 100.0)
    b2 = (x > 100.0) & (x <= 1000.0)
    lnx = jnp.log(x)
    c = jnp.where(b0, 1.5e-4, jnp.where(b1, 0.035, jnp.where(b2, 0.15, -1.0)))
    d = jnp.where(b0, 2.52, jnp.where(b1, 1.63, jnp.where(b2, 1.30, 0.0)))
    cond_ref[...] = c * jnp.exp(d * lnx)


def _tc_cond(frequency):
    f2 = frequency.reshape(ROWS, LANES)
    cond = pl.pallas_call(
        _tc_body,
        grid=(GRID,),
        in_specs=[pl.BlockSpec((BR, LANES), lambda i: (i, 0))],
        out_specs=pl.BlockSpec((BR, LANES), lambda i: (i, 0)),
        out_shape=jax.ShapeDtypeStruct((ROWS, LANES), jnp.float32),
    )(f2)
    return cond.reshape(N)


def kernel(frequency):
    ftab, btab = _table_inputs()
    rel = _build_sc_rel()(frequency, ftab, btab)
    cond = _tc_cond(frequency)
    return rel, cond


# speedup vs baseline: 1.7129x; 1.7129x over previous
import functools

import jax
import jax.numpy as jnp
from jax import lax
from jax.experimental import pallas as pl
from jax.experimental.pallas import tpu as pltpu
from jax.experimental.pallas import tpu_sc as plsc

N = 16777216
NC, NS, L = 2, 16, 16
NW = NC * NS
PW = N // NW
C = 16384
NCHUNK = PW // C

_LN2 = 0.6931471805599453

_A = (3.0, 15.0, 30.0, -1.0)
_B = (0.0, -0.1, -0.4, 0.0)

_BAND_BY_SLOT = (3, 0, 0, 0, 3, 1, 1, 3, 2, 2, 2, 3, 0, 1, 2, 3)
_NEG = float("-inf")
_THR_TAB = (_NEG, 1.0e9, _NEG, _NEG, _NEG, 1.0e10, _NEG, _NEG,
            100000006144.0, _NEG, _NEG, 1.0e12, _NEG, _NEG, _NEG, _NEG)
_BASE_TAB = (13, 0, 10, 10, 10, 3, 11, 11, 6, 12, 12, 9, 13, 0, 0, 0)

_P4 = (-159.80915647634861, 5.371138987534238, -3.6996336354567507,
       1.4905116583281666, -0.2501516357927904)


def _table_inputs():
    rows = [
        _THR_TAB,
        [_A[i] for i in _BAND_BY_SLOT],
        [_B[i] * _LN2 for i in _BAND_BY_SLOT],
    ]
    return (jnp.asarray(rows, dtype=jnp.float32),
            jnp.asarray(_BASE_TAB, dtype=jnp.int32))


def _take(vec, idx):
    return jnp.take_along_axis(vec, idx, axis=0, mode="promise_in_bounds")


def _eval_vec(f, thr_v, base_v, a_v, bln_v):
    bits = lax.bitcast_convert_type(f, jnp.int32)

    idx_bin = (bits >> 23) - 155
    thr = _take(thr_v, idx_bin)
    one = jnp.ones_like(idx_bin)
    g = jnp.where(f > thr, 2 * one, jnp.where(f >= thr, one, 0 * one))
    slot = _take(base_v, idx_bin) + g
    a = _take(a_v, slot)
    bln = _take(bln_v, slot)

    ebr = (bits + 0x00400000) >> 23
    scale = lax.bitcast_convert_type((254 - ebr) << 23, jnp.float32)
    m = f * scale
    p = jnp.float32(_P4[4])
    p = p * m + jnp.float32(_P4[3])
    p = p * m + jnp.float32(_P4[2])
    p = p * m + jnp.float32(_P4[1])
    p = p * m + jnp.float32(_P4[0])
    lg2 = p + ebr.astype(jnp.float32)

    return a * jnp.exp(bln * lg2)


@functools.cache
def _build_sc_rel():
    @functools.partial(
        pl.kernel,
        out_type=jax.ShapeDtypeStruct((N,), jnp.float32),
        mesh=plsc.VectorSubcoreMesh(
            core_axis_name="c", subcore_axis_name="s",
            num_cores=NC, num_subcores=NS),
        scratch_types=[
            pltpu.VMEM((3, L), jnp.float32), pltpu.VMEM((L,), jnp.int32),
            pltpu.VMEM((C,), jnp.float32), pltpu.VMEM((C,), jnp.float32),
            pltpu.VMEM((C,), jnp.float32), pltpu.VMEM((C,), jnp.float32),
            pltpu.SemaphoreType.DMA, pltpu.SemaphoreType.DMA,
            pltpu.SemaphoreType.DMA, pltpu.SemaphoreType.DMA,
        ],
    )
    def _sc_rel(freq_hbm, ftab_hbm, btab_hbm, rel_hbm,
                ftab_v, btab_v, in0, in1, rel0, rel1,
                isem0, isem1, osem0, osem1):
        wid = lax.axis_index("s") * NC + lax.axis_index("c")
        base0 = wid * PW

        def in_slice(k):
            return freq_hbm.at[pl.ds(base0 + k * C, C)]

        pltpu.async_copy(in_slice(0), in0, isem0)
        pltpu.async_copy(in_slice(1), in1, isem1)
        pltpu.sync_copy(ftab_hbm, ftab_v)
        pltpu.sync_copy(btab_hbm, btab_v)

        thr_v = ftab_v[0]
        a_v = ftab_v[1]
        bln_v = ftab_v[2]
        base_v = btab_v[...]

        def compute(in_v, rel_v):
            @plsc.parallel_loop(0, C, L)
            def _(i):
                rel_v[pl.ds(i, L)] = _eval_vec(
                    in_v[pl.ds(i, L)], thr_v, base_v, a_v, bln_v)

        def half(kk, k, in_v, rel_v, isem, osem):
            base = base0 + k * C
            pltpu.make_async_copy(in_slice(k), in_v, isem).wait()

            @pl.when(kk > 0)
            def _():
                pltpu.make_async_copy(
                    rel_v, rel_hbm.at[pl.ds(base, C)], osem).wait()

            compute(in_v, rel_v)
            pltpu.async_copy(rel_v, rel_hbm.at[pl.ds(base, C)], osem)

            @pl.when(kk < NCHUNK // 2 - 1)
            def _():
                pltpu.async_copy(in_slice(k + 2), in_v, isem)

        def pair_body(kk, carry):
            half(kk, 2 * kk, in0, rel0, isem0, osem0)
            half(kk, 2 * kk + 1, in1, rel1, isem1, osem1)
            return carry

        lax.fori_loop(0, NCHUNK // 2, pair_body, 0)

        last0 = base0 + (NCHUNK - 2) * C
        last1 = base0 + (NCHUNK - 1) * C
        pltpu.make_async_copy(rel0, rel_hbm.at[pl.ds(last0, C)], osem0).wait()
        pltpu.make_async_copy(rel1, rel_hbm.at[pl.ds(last1, C)], osem1).wait()

    return _sc_rel


BN = 524288
TC_GRID = N // BN


def _tc_body(f_ref, cond_ref):
    f = f_ref[...]
    x = f * jnp.float32(1e-9)
    b0 = (x >= 1.0) & (x < 10.0)
    b1 = (x > 10.0) & (x < 100.0)
    b2 = (x > 100.0) & (x <= 1000.0)
    lnx = jnp.log(x)
    c = jnp.where(b0, 1.5e-4, jnp.where(b1, 0.035, jnp.where(b2, 0.15, -1.0)))
    d = jnp.where(b0, 2.52, jnp.where(b1, 1.63, jnp.where(b2, 1.30, 0.0)))
    cond_ref[...] = c * jnp.exp(d * lnx)


def _tc_cond(frequency):
    return pl.pallas_call(
        _tc_body,
        grid=(TC_GRID,),
        in_specs=[pl.BlockSpec((BN,), lambda i: (i,))],
        out_specs=pl.BlockSpec((BN,), lambda i: (i,)),
        out_shape=jax.ShapeDtypeStruct((N,), jnp.float32),
    )(frequency)


def kernel(frequency):
    ftab, btab = _table_inputs()
    rel = _build_sc_rel()(frequency, ftab, btab)
    cond = _tc_cond(frequency)
    return rel, cond


# speedup vs baseline: 2.8898x; 1.6871x over previous
import functools

import jax
import jax.numpy as jnp
from jax import lax
from jax.experimental import pallas as pl
from jax.experimental.pallas import tpu as pltpu
from jax.experimental.pallas import tpu_sc as plsc

N = 16777216
NSC = 7340032
NC, NS, L = 2, 16, 16
NW = NC * NS
PW = NSC // NW
C = 16384
NCHUNK = PW // C

_LN2 = 0.6931471805599453

_A = (3.0, 15.0, 30.0, -1.0)
_B = (0.0, -0.1, -0.4, 0.0)

_BAND_BY_SLOT = (3, 0, 0, 0, 3, 1, 1, 3, 2, 2, 2, 3, 0, 1, 2, 3)
_NEG = float("-inf")
_THR_TAB = (_NEG, 1.0e9, _NEG, _NEG, _NEG, 1.0e10, _NEG, _NEG,
            100000006144.0, _NEG, _NEG, 1.0e12, _NEG, _NEG, _NEG, _NEG)
_BASE_TAB = (13, 0, 10, 10, 10, 3, 11, 11, 6, 12, 12, 9, 13, 0, 0, 0)

_P4 = (-159.80915647634861, 5.371138987534238, -3.6996336354567507,
       1.4905116583281666, -0.2501516357927904)


def _table_inputs():
    rows = [
        _THR_TAB,
        [_A[i] for i in _BAND_BY_SLOT],
        [_B[i] * _LN2 for i in _BAND_BY_SLOT],
    ]
    return (jnp.asarray(rows, dtype=jnp.float32),
            jnp.asarray(_BASE_TAB, dtype=jnp.int32))


def _take(vec, idx):
    return jnp.take_along_axis(vec, idx, axis=0, mode="promise_in_bounds")


def _eval_vec(f, thr_v, base_v, a_v, bln_v):
    bits = lax.bitcast_convert_type(f, jnp.int32)

    idx_bin = (bits >> 23) - 155
    thr = _take(thr_v, idx_bin)
    one = jnp.ones_like(idx_bin)
    g = jnp.where(f > thr, 2 * one, jnp.where(f >= thr, one, 0 * one))
    slot = _take(base_v, idx_bin) + g
    a = _take(a_v, slot)
    bln = _take(bln_v, slot)

    ebr = (bits + 0x00400000) >> 23
    scale = lax.bitcast_convert_type((254 - ebr) << 23, jnp.float32)
    m = f * scale
    p = jnp.float32(_P4[4])
    p = p * m + jnp.float32(_P4[3])
    p = p * m + jnp.float32(_P4[2])
    p = p * m + jnp.float32(_P4[1])
    p = p * m + jnp.float32(_P4[0])
    lg2 = p + ebr.astype(jnp.float32)

    return a * jnp.exp(bln * lg2)


@functools.cache
def _build_sc_rel():
    @functools.partial(
        pl.kernel,
        out_type=jax.ShapeDtypeStruct((NSC,), jnp.float32),
        mesh=plsc.VectorSubcoreMesh(
            core_axis_name="c", subcore_axis_name="s",
            num_cores=NC, num_subcores=NS),
        scratch_types=[
            pltpu.VMEM((3, L), jnp.float32), pltpu.VMEM((L,), jnp.int32),
            pltpu.VMEM((C,), jnp.float32), pltpu.VMEM((C,), jnp.float32),
            pltpu.VMEM((C,), jnp.float32), pltpu.VMEM((C,), jnp.float32),
            pltpu.SemaphoreType.DMA, pltpu.SemaphoreType.DMA,
            pltpu.SemaphoreType.DMA, pltpu.SemaphoreType.DMA,
        ],
    )
    def _sc_rel(freq_hbm, ftab_hbm, btab_hbm, rel_hbm,
                ftab_v, btab_v, in0, in1, rel0, rel1,
                isem0, isem1, osem0, osem1):
        wid = lax.axis_index("s") * NC + lax.axis_index("c")
        base0 = wid * PW

        def in_slice(k):
            return freq_hbm.at[pl.ds(base0 + k * C, C)]

        pltpu.async_copy(in_slice(0), in0, isem0)
        pltpu.async_copy(in_slice(1), in1, isem1)
        pltpu.sync_copy(ftab_hbm, ftab_v)
        pltpu.sync_copy(btab_hbm, btab_v)

        thr_v = ftab_v[0]
        a_v = ftab_v[1]
        bln_v = ftab_v[2]
        base_v = btab_v[...]

        def compute(in_v, rel_v):
            @plsc.parallel_loop(0, C, L)
            def _(i):
                rel_v[pl.ds(i, L)] = _eval_vec(
                    in_v[pl.ds(i, L)], thr_v, base_v, a_v, bln_v)

        def half(kk, k, in_v, rel_v, isem, osem):
            base = base0 + k * C
            pltpu.make_async_copy(in_slice(k), in_v, isem).wait()

            @pl.when(kk > 0)
            def _():
                pltpu.make_async_copy(
                    rel_v, rel_hbm.at[pl.ds(base, C)], osem).wait()

            compute(in_v, rel_v)
            pltpu.async_copy(rel_v, rel_hbm.at[pl.ds(base, C)], osem)

            @pl.when(kk < NCHUNK // 2 - 1)
            def _():
                pltpu.async_copy(in_slice(k + 2), in_v, isem)

        def pair_body(kk, carry):
            half(kk, 2 * kk, in0, rel0, isem0, osem0)
            half(kk, 2 * kk + 1, in1, rel1, isem1, osem1)
            return carry

        lax.fori_loop(0, NCHUNK // 2, pair_body, 0)

        last0 = base0 + (NCHUNK - 2) * C
        last1 = base0 + (NCHUNK - 1) * C
        pltpu.make_async_copy(rel0, rel_hbm.at[pl.ds(last0, C)], osem0).wait()
        pltpu.make_async_copy(rel1, rel_hbm.at[pl.ds(last1, C)], osem1).wait()

    return _sc_rel


BN = 524288
TC_GRID = N // BN
REL_TAIL_BLOCK = NSC // BN


def _tc_body(f_ref, cond_ref, rel_ref):
    f = f_ref[...]
    x = f * jnp.float32(1e-9)
    b0 = (x >= 1.0) & (x < 10.0)
    b1 = (x > 10.0) & (x < 100.0)
    b2 = (x > 100.0) & (x <= 1000.0)
    lnx = jnp.log(x)
    c = jnp.where(b0, 1.5e-4, jnp.where(b1, 0.035, jnp.where(b2, 0.15, -1.0)))
    d = jnp.where(b0, 2.52, jnp.where(b1, 1.63, jnp.where(b2, 1.30, 0.0)))
    cond_ref[...] = c * jnp.exp(d * lnx)

    @pl.when(pl.program_id(0) >= REL_TAIL_BLOCK)
    def _():
        a = jnp.where(b0, 3.0, jnp.where(b1, 15.0, jnp.where(b2, 30.0, -1.0)))
        b = jnp.where(b1, -0.1, jnp.where(b2, -0.4, 0.0))
        rel_ref[...] = a * jnp.exp(b * lnx)


def _tc_cond_and_rel_tail(frequency):
    return pl.pallas_call(
        _tc_body,
        grid=(TC_GRID,),
        in_specs=[pl.BlockSpec((BN,), lambda i: (i,))],
        out_specs=[pl.BlockSpec((BN,), lambda i: (i,)),
                   pl.BlockSpec((BN,), lambda i: (i,))],
        out_shape=[jax.ShapeDtypeStruct((N,), jnp.float32),
                   jax.ShapeDtypeStruct((N,), jnp.float32)],
    )(frequency)


def kernel(frequency):
    ftab, btab = _table_inputs()
    rel_head = _build_sc_rel()(frequency, ftab, btab)
    cond, rel = _tc_cond_and_rel_tail(frequency)
    rel = lax.dynamic_update_slice(rel, rel_head, (0,))
    return rel, cond


# speedup vs baseline: 3.0143x; 1.0431x over previous
import functools

import jax
import jax.numpy as jnp
from jax import lax
from jax.experimental import pallas as pl
from jax.experimental.pallas import tpu as pltpu
from jax.experimental.pallas import tpu_sc as plsc

N = 16777216
NSC = 6815744
NC, NS, L = 2, 16, 16
NW = NC * NS
PW = NSC // NW
C = 8192
NCHUNK = PW // C

_LN2 = 0.6931471805599453

_A = (3.0, 15.0, 30.0, -1.0)
_B = (0.0, -0.1, -0.4, 0.0)

_BAND_BY_SLOT = (3, 0, 0, 0, 3, 1, 1, 3, 2, 2, 2, 3, 0, 1, 2, 3)
_NEG = float("-inf")
_THR_TAB = (_NEG, 1.0e9, _NEG, _NEG, _NEG, 1.0e10, _NEG, _NEG,
            100000006144.0, _NEG, _NEG, 1.0e12, _NEG, _NEG, _NEG, _NEG)
_BASE_TAB = (13, 0, 10, 10, 10, 3, 11, 11, 6, 12, 12, 9, 13, 0, 0, 0)

_P3 = (-159.44619948198554, 4.014290052244012, -1.8301970080156211,
       0.36482929726059316)


def _table_inputs():
    rows = [
        _THR_TAB,
        [_A[i] for i in _BAND_BY_SLOT],
        [_B[i] * _LN2 for i in _BAND_BY_SLOT],
    ]
    return (jnp.asarray(rows, dtype=jnp.float32),
            jnp.asarray(_BASE_TAB, dtype=jnp.int32))


def _take(vec, idx):
    return jnp.take_along_axis(vec, idx, axis=0, mode="promise_in_bounds")


def _eval_vec(f, thr_v, base_v, a_v, bln_v):
    bits = lax.bitcast_convert_type(f, jnp.int32)

    idx_bin = (bits >> 23) - 155
    thr = _take(thr_v, idx_bin)
    one = jnp.ones_like(idx_bin)
    g = jnp.where(f > thr, 2 * one, jnp.where(f >= thr, one, 0 * one))
    slot = _take(base_v, idx_bin) + g
    a = _take(a_v, slot)
    bln = _take(bln_v, slot)

    ebr = (bits + 0x00400000) >> 23
    scale = lax.bitcast_convert_type((254 - ebr) << 23, jnp.float32)
    m = f * scale
    p = jnp.float32(_P3[3])
    p = p * m + jnp.float32(_P3[2])
    p = p * m + jnp.float32(_P3[1])
    p = p * m + jnp.float32(_P3[0])
    lg2 = p + ebr.astype(jnp.float32)

    return a * jnp.exp(bln * lg2)


@functools.cache
def _build_sc_rel():
    @functools.partial(
        pl.kernel,
        out_type=jax.ShapeDtypeStruct((NSC,), jnp.float32),
        mesh=plsc.VectorSubcoreMesh(
            core_axis_name="c", subcore_axis_name="s",
            num_cores=NC, num_subcores=NS),
        scratch_types=[
            pltpu.VMEM((3, L), jnp.float32), pltpu.VMEM((L,), jnp.int32),
            pltpu.VMEM((C,), jnp.float32), pltpu.VMEM((C,), jnp.float32),
            pltpu.VMEM((C,), jnp.float32), pltpu.VMEM((C,), jnp.float32),
            pltpu.SemaphoreType.DMA, pltpu.SemaphoreType.DMA,
            pltpu.SemaphoreType.DMA, pltpu.SemaphoreType.DMA,
        ],
    )
    def _sc_rel(freq_hbm, ftab_hbm, btab_hbm, rel_hbm,
                ftab_v, btab_v, in0, in1, rel0, rel1,
                isem0, isem1, osem0, osem1):
        wid = lax.axis_index("s") * NC + lax.axis_index("c")
        base0 = wid * PW

        def in_slice(k):
            return freq_hbm.at[pl.ds(base0 + k * C, C)]

        pltpu.async_copy(in_slice(0), in0, isem0)
        pltpu.async_copy(in_slice(1), in1, isem1)
        pltpu.sync_copy(ftab_hbm, ftab_v)
        pltpu.sync_copy(btab_hbm, btab_v)

        thr_v = ftab_v[0]
        a_v = ftab_v[1]
        bln_v = ftab_v[2]
        base_v = btab_v[...]

        def compute(in_v, rel_v):
            @plsc.parallel_loop(0, C, L)
            def _(i):
                rel_v[pl.ds(i, L)] = _eval_vec(
                    in_v[pl.ds(i, L)], thr_v, base_v, a_v, bln_v)

        def half(kk, k, in_v, rel_v, isem, osem):
            base = base0 + k * C
            pltpu.make_async_copy(in_slice(k), in_v, isem).wait()

            @pl.when(kk > 0)
            def _():
                pltpu.make_async_copy(
                    rel_v, rel_hbm.at[pl.ds(base, C)], osem).wait()

            compute(in_v, rel_v)
            pltpu.async_copy(rel_v, rel_hbm.at[pl.ds(base, C)], osem)

            @pl.when(kk < NCHUNK // 2 - 1)
            def _():
                pltpu.async_copy(in_slice(k + 2), in_v, isem)

        def pair_body(kk, carry):
            half(kk, 2 * kk, in0, rel0, isem0, osem0)
            half(kk, 2 * kk + 1, in1, rel1, isem1, osem1)
            return carry

        lax.fori_loop(0, NCHUNK // 2, pair_body, 0)

        last0 = base0 + (NCHUNK - 2) * C
        last1 = base0 + (NCHUNK - 1) * C
        pltpu.make_async_copy(rel0, rel_hbm.at[pl.ds(last0, C)], osem0).wait()
        pltpu.make_async_copy(rel1, rel_hbm.at[pl.ds(last1, C)], osem1).wait()

    return _sc_rel


BN = 524288
TC_GRID = N // BN
REL_TAIL_BLOCK = NSC // BN


def _tc_body(f_ref, cond_ref, rel_ref):
    f = f_ref[...]
    x = f * jnp.float32(1e-9)
    b0 = (x >= 1.0) & (x < 10.0)
    b1 = (x > 10.0) & (x < 100.0)
    b2 = (x > 100.0) & (x <= 1000.0)
    lnx = jnp.log(x)
    c = jnp.where(b0, 1.5e-4, jnp.where(b1, 0.035, jnp.where(b2, 0.15, -1.0)))
    d = jnp.where(b0, 2.52, jnp.where(b1, 1.63, jnp.where(b2, 1.30, 0.0)))
    cond_ref[...] = c * jnp.exp(d * lnx)

    @pl.when(pl.program_id(0) >= REL_TAIL_BLOCK)
    def _():
        a = jnp.where(b0, 3.0, jnp.where(b1, 15.0, jnp.where(b2, 30.0, -1.0)))
        b = jnp.where(b1, -0.1, jnp.where(b2, -0.4, 0.0))
        rel_ref[...] = a * jnp.exp(b * lnx)


def _tc_cond_and_rel_tail(frequency):
    return pl.pallas_call(
        _tc_body,
        grid=(TC_GRID,),
        in_specs=[pl.BlockSpec((BN,), lambda i: (i,))],
        out_specs=[pl.BlockSpec((BN,), lambda i: (i,)),
                   pl.BlockSpec((BN,), lambda i: (i,))],
        out_shape=[jax.ShapeDtypeStruct((N,), jnp.float32),
                   jax.ShapeDtypeStruct((N,), jnp.float32)],
    )(frequency)


def kernel(frequency):
    ftab, btab = _table_inputs()
    rel_head = _build_sc_rel()(frequency, ftab, btab)
    cond, rel = _tc_cond_and_rel_tail(frequency)
    rel = lax.dynamic_update_slice(rel, rel_head, (0,))
    return rel, cond


# speedup vs baseline: 3.1878x; 1.0576x over previous
import functools

import jax
import jax.numpy as jnp
from jax import lax
from jax.experimental import pallas as pl
from jax.experimental.pallas import tpu as pltpu
from jax.experimental.pallas import tpu_sc as plsc

N = 16777216
NSC = 6815744
NC, NS, L = 2, 16, 16
NW = NC * NS
PW = NSC // NW
C = 8192
NCHUNK = PW // C

_LN2 = 0.6931471805599453

_A = (3.0, 15.0, 30.0, -1.0)
_B = (0.0, -0.1, -0.4, 0.0)

_BAND_BY_SLOT = (3, 0, 0, 0, 3, 1, 1, 3, 2, 2, 2, 3, 0, 1, 2, 3)
_NEG = float("-inf")
_THR_TAB = (_NEG, 1.0e9, _NEG, _NEG, _NEG, 1.0e10, _NEG, _NEG,
            100000006144.0, _NEG, _NEG, 1.0e12, _NEG, _NEG, _NEG, _NEG)
_BASE_TAB = (13, 0, 10, 10, 10, 3, 11, 11, 6, 12, 12, 9, 13, 0, 0, 0)

_P3 = (-159.44619948198554, 4.014290052244012, -1.8301970080156211,
       0.36482929726059316)


def _table_inputs():
    rows = [
        _THR_TAB,
        [_A[i] for i in _BAND_BY_SLOT],
        [_B[i] * _LN2 for i in _BAND_BY_SLOT],
    ]
    return (jnp.asarray(rows, dtype=jnp.float32),
            jnp.asarray(_BASE_TAB, dtype=jnp.int32))


def _take(vec, idx):
    return jnp.take_along_axis(vec, idx, axis=0, mode="promise_in_bounds")


def _eval_vec(f, thr_v, base_v, a_v, bln_v):
    bits = lax.bitcast_convert_type(f, jnp.int32)

    idx_bin = (bits >> 23) - 155
    thr = _take(thr_v, idx_bin)
    one = jnp.ones_like(idx_bin)
    g = jnp.where(f > thr, 2 * one, jnp.where(f >= thr, one, 0 * one))
    slot = _take(base_v, idx_bin) + g
    a = _take(a_v, slot)
    bln = _take(bln_v, slot)

    ebr = (bits + 0x00400000) >> 23
    scale = lax.bitcast_convert_type((254 - ebr) << 23, jnp.float32)
    m = f * scale
    p = jnp.float32(_P3[3])
    p = p * m + jnp.float32(_P3[2])
    p = p * m + jnp.float32(_P3[1])
    p = p * m + jnp.float32(_P3[0])
    lg2 = p + ebr.astype(jnp.float32)

    return a * jnp.exp(bln * lg2)


@functools.cache
def _build_sc_rel():
    @functools.partial(
        pl.kernel,
        out_type=jax.ShapeDtypeStruct((NSC,), jnp.float32),
        mesh=plsc.VectorSubcoreMesh(
            core_axis_name="c", subcore_axis_name="s",
            num_cores=NC, num_subcores=NS),
        scratch_types=[
            pltpu.VMEM((3, L), jnp.float32), pltpu.VMEM((L,), jnp.int32),
            pltpu.VMEM((C,), jnp.float32), pltpu.VMEM((C,), jnp.float32),
            pltpu.VMEM((C,), jnp.float32), pltpu.VMEM((C,), jnp.float32),
            pltpu.SemaphoreType.DMA, pltpu.SemaphoreType.DMA,
            pltpu.SemaphoreType.DMA, pltpu.SemaphoreType.DMA,
        ],
    )
    def _sc_rel(freq_hbm, ftab_hbm, btab_hbm, rel_hbm,
                ftab_v, btab_v, in0, in1, rel0, rel1,
                isem0, isem1, osem0, osem1):
        wid = lax.axis_index("s") * NC + lax.axis_index("c")
        base0 = wid * PW

        def in_slice(k):
            return freq_hbm.at[pl.ds(base0 + k * C, C)]

        pltpu.async_copy(in_slice(0), in0, isem0)
        pltpu.async_copy(in_slice(1), in1, isem1)
        pltpu.sync_copy(ftab_hbm, ftab_v)
        pltpu.sync_copy(btab_hbm, btab_v)

        thr_v = ftab_v[0]
        a_v = ftab_v[1]
        bln_v = ftab_v[2]
        base_v = btab_v[...]

        def compute(in_v, rel_v):
            @plsc.parallel_loop(0, C, L)
            def _(i):
                rel_v[pl.ds(i, L)] = _eval_vec(
                    in_v[pl.ds(i, L)], thr_v, base_v, a_v, bln_v)

        def half(kk, k, in_v, rel_v, isem, osem):
            base = base0 + k * C
            pltpu.make_async_copy(in_slice(k), in_v, isem).wait()

            @pl.when(kk > 0)
            def _():
                pltpu.make_async_copy(
                    rel_v, rel_hbm.at[pl.ds(base, C)], osem).wait()

            compute(in_v, rel_v)
            pltpu.async_copy(rel_v, rel_hbm.at[pl.ds(base, C)], osem)

            @pl.when(kk < NCHUNK // 2 - 1)
            def _():
                pltpu.async_copy(in_slice(k + 2), in_v, isem)

        def pair_body(kk, carry):
            half(kk, 2 * kk, in0, rel0, isem0, osem0)
            half(kk, 2 * kk + 1, in1, rel1, isem1, osem1)
            return carry

        lax.fori_loop(0, NCHUNK // 2, pair_body, 0)

        last0 = base0 + (NCHUNK - 2) * C
        last1 = base0 + (NCHUNK - 1) * C
        pltpu.make_async_copy(rel0, rel_hbm.at[pl.ds(last0, C)], osem0).wait()
        pltpu.make_async_copy(rel1, rel_hbm.at[pl.ds(last1, C)], osem1).wait()

    return _sc_rel


BN = 524288
TC_GRID = N // BN
REL_TAIL_BLOCK = NSC // BN


def _tc_body(f_ref, cond_ref, rel_ref):
    f = f_ref[...]
    x = f * jnp.float32(1e-9)
    b0 = (x >= 1.0) & (x < 10.0)
    b1 = (x > 10.0) & (x < 100.0)
    b2 = (x > 100.0) & (x <= 1000.0)
    lnx = jnp.log(x)
    c = jnp.where(b0, 1.5e-4, jnp.where(b1, 0.035, jnp.where(b2, 0.15, -1.0)))
    d = jnp.where(b0, 2.52, jnp.where(b1, 1.63, jnp.where(b2, 1.30, 0.0)))
    cond_ref[...] = c * jnp.exp(d * lnx)

    @pl.when(pl.program_id(0) >= REL_TAIL_BLOCK)
    def _():
        a = jnp.where(b0, 3.0, jnp.where(b1, 15.0, jnp.where(b2, 30.0, -1.0)))
        b = jnp.where(b1, -0.1, jnp.where(b2, -0.4, 0.0))
        rel_ref[...] = a * jnp.exp(b * lnx)


def _tc_cond_and_rel_tail(frequency):
    return pl.pallas_call(
        _tc_body,
        grid=(TC_GRID,),
        in_specs=[pl.BlockSpec((BN,), lambda i: (i,))],
        out_specs=[pl.BlockSpec((BN,), lambda i: (i,)),
                   pl.BlockSpec(
                       (BN,),
                       lambda i: (jnp.maximum(i, REL_TAIL_BLOCK),))],
        out_shape=[jax.ShapeDtypeStruct((N,), jnp.float32),
                   jax.ShapeDtypeStruct((N,), jnp.float32)],
    )(frequency)


def kernel(frequency):
    ftab, btab = _table_inputs()
    rel_head = _build_sc_rel()(frequency, ftab, btab)
    cond, rel = _tc_cond_and_rel_tail(frequency)
    rel = lax.dynamic_update_slice(rel, rel_head, (0,))
    return rel, cond
